# baseline (speedup 1.0000x reference)
import functools

import jax
import jax.numpy as jnp
from jax import lax
from jax.experimental import pallas as pl
from jax.experimental.pallas import tpu as pltpu
from jax.experimental.pallas import tpu_sc as plsc

ROWS = 32768
CH = 2048
L = 16
NC = 2
NS = 16
NW = NC * NS
ROWS_PER_W = ROWS // NW
R = 8
NCHUNK = ROWS_PER_W // R
G = CH // L

_mesh = plsc.VectorSubcoreMesh(core_axis_name="c", subcore_axis_name="s")


@functools.partial(
    pl.kernel,
    mesh=_mesh,
    out_type=jax.ShapeDtypeStruct((ROWS * CH,), jnp.float32),
    scratch_types=[
        pltpu.VMEM((CH,), jnp.int32),
        pltpu.VMEM((R * CH,), jnp.float32),
        pltpu.VMEM((R * CH,), jnp.float32),
    ],
    compiler_params=pltpu.CompilerParams(needs_layout_passes=False),
)
def _permute(in_hbm, perm_hbm, out_hbm, perm_v, inbuf, outbuf):
    wid = lax.axis_index("s") * NC + lax.axis_index("c")
    base = wid * ROWS_PER_W * CH
    pltpu.sync_copy(perm_hbm, perm_v)

    def chunk_body(c, carry):
        off = base + c * (R * CH)
        pltpu.sync_copy(in_hbm.at[pl.ds(off, R * CH)], inbuf)

        def group_body(g, carry2):
            idx = perm_v[pl.ds(g * L, L)]
            for r in range(R):
                v = plsc.load_gather(inbuf, [idx + (r * CH)])
                outbuf[pl.ds(g * L + r * CH, L)] = v
            return carry2

        lax.fori_loop(0, G, group_body, 0)
        pltpu.sync_copy(outbuf, out_hbm.at[pl.ds(off, R * CH)])
        return carry

    lax.fori_loop(0, NCHUNK, chunk_body, 0)


def kernel(input, perm):
    out_flat = _permute(input.reshape(-1), perm.astype(jnp.int32))
    return out_flat.reshape(ROWS, CH)


# speedup vs baseline: 1.2449x; 1.2449x over previous
import functools

import jax
import jax.numpy as jnp
from jax import lax
from jax.experimental import pallas as pl
from jax.experimental.pallas import tpu as pltpu
from jax.experimental.pallas import tpu_sc as plsc

ROWS = 32768
CH = 2048
L = 16
NC = 2
NS = 16
NW = NC * NS
ROWS_PER_W = ROWS // NW
R = 8
NCHUNK = ROWS_PER_W // R
NPAIR = NCHUNK // 2
G = CH // L
CB = R * CH

_mesh = plsc.VectorSubcoreMesh(core_axis_name="c", subcore_axis_name="s")


@functools.partial(
    pl.kernel,
    mesh=_mesh,
    out_type=jax.ShapeDtypeStruct((ROWS * CH,), jnp.float32),
    scratch_types=[
        pltpu.VMEM((CH,), jnp.int32),
        pltpu.VMEM((CB,), jnp.float32),
        pltpu.VMEM((CB,), jnp.float32),
        pltpu.VMEM((CB,), jnp.float32),
        pltpu.VMEM((CB,), jnp.float32),
        pltpu.SemaphoreType.DMA,
        pltpu.SemaphoreType.DMA,
        pltpu.SemaphoreType.DMA,
        pltpu.SemaphoreType.DMA,
    ],
    compiler_params=pltpu.CompilerParams(needs_layout_passes=False),
)
def _permute(in_hbm, perm_hbm, out_hbm, perm_v,
             in0, in1, out0, out1, si0, si1, so0, so1):
    wid = lax.axis_index("s") * NC + lax.axis_index("c")
    base = wid * ROWS_PER_W * CH
    pltpu.sync_copy(perm_hbm, perm_v)

    ins, outs = [in0, in1], [out0, out1]
    sis, sos = [si0, si1], [so0, so1]

    def in_copy(n, b):
        return pltpu.make_async_copy(
            in_hbm.at[pl.ds(base + n * CB, CB)], ins[b], sis[b])

    def out_copy(n, b):
        return pltpu.make_async_copy(
            outs[b], out_hbm.at[pl.ds(base + n * CB, CB)], sos[b])

    def compute(b):
        inbuf, outbuf = ins[b], outs[b]

        def group_body(g, carry):
            idx = perm_v[pl.ds(g * L, L)]
            for r in range(R):
                v = plsc.load_gather(inbuf, [idx + (r * CH)])
                outbuf[pl.ds(g * L + r * CH, L)] = v
            return carry

        lax.fori_loop(0, G, group_body, 0, unroll=2)

    in_copy(0, 0).start()

    def pair_body(c, carry):
        n0 = 2 * c
        in_copy(n0 + 1, 1).start()
        in_copy(n0, 0).wait()

        @pl.when(c > 0)
        def _():
            out_copy(n0 - 2, 0).wait()

        compute(0)
        out_copy(n0, 0).start()

        @pl.when(c < NPAIR - 1)
        def _():
            in_copy(n0 + 2, 0).start()

        in_copy(n0 + 1, 1).wait()

        @pl.when(c > 0)
        def _():
            out_copy(n0 - 1, 1).wait()

        compute(1)
        out_copy(n0 + 1, 1).start()
        return carry

    lax.fori_loop(0, NPAIR, pair_body, 0)
    out_copy(NCHUNK - 2, 0).wait()
    out_copy(NCHUNK - 1, 1).wait()


def kernel(input, perm):
    out_flat = _permute(input.reshape(-1), perm.astype(jnp.int32))
    return out_flat.reshape(ROWS, CH)


# speedup vs baseline: 2.1160x; 1.6998x over previous
import functools

import jax
import jax.numpy as jnp
from jax import lax
from jax.experimental import pallas as pl
from jax.experimental.pallas import tpu as pltpu
from jax.experimental.pallas import tpu_sc as plsc

ROWS = 32768
CH = 2048
L = 16
NC = 2
NS = 16
NW = NC * NS
ROWS_PER_W = ROWS // NW
R = 8
NCHUNK = ROWS_PER_W // R
NPAIR = NCHUNK // 2
G = CH // L
CB = R * CH

_mesh = plsc.VectorSubcoreMesh(core_axis_name="c", subcore_axis_name="s")


@functools.partial(
    pl.kernel,
    mesh=_mesh,
    out_type=jax.ShapeDtypeStruct((ROWS * CH,), jnp.float32),
    scratch_types=[
        pltpu.VMEM((CH,), jnp.int32),
        pltpu.VMEM((CB,), jnp.float32),
        pltpu.VMEM((CB,), jnp.float32),
        pltpu.VMEM((CB,), jnp.float32),
        pltpu.VMEM((CB,), jnp.float32),
        pltpu.SemaphoreType.DMA,
        pltpu.SemaphoreType.DMA,
        pltpu.SemaphoreType.DMA,
        pltpu.SemaphoreType.DMA,
    ],
    compiler_params=pltpu.CompilerParams(needs_layout_passes=False),
)
def _permute(in_hbm, pidx_hbm, out_hbm, pidx_v,
             in0, in1, out0, out1, si0, si1, so0, so1):
    wid = lax.axis_index("s") * NC + lax.axis_index("c")
    base = wid * ROWS_PER_W * CH
    pltpu.sync_copy(pidx_hbm, pidx_v)

    ins, outs = [in0, in1], [out0, out1]
    sis, sos = [si0, si1], [so0, so1]

    def in_copy(n, b):
        return pltpu.make_async_copy(
            in_hbm.at[pl.ds(base + n * CB, CB)], ins[b], sis[b])

    def out_copy(n, b):
        return pltpu.make_async_copy(
            outs[b], out_hbm.at[pl.ds(base + n * CB, CB)], sos[b])

    def compute(b):
        inbuf, outbuf = ins[b], outs[b]

        def group_body(g, carry):
            q0 = (g >> 3) * 1024 + (g & 7) * L
            idx = pidx_v[pl.ds(g * L, L)]
            for r in range(R):
                v = plsc.load_gather(inbuf, [idx + (r * 128)])
                outbuf[pl.ds(q0 + r * 128, L)] = v
            return carry

        lax.fori_loop(0, G, group_body, 0, unroll=2)

    in_copy(0, 0).start()

    def pair_body(c, carry):
        n0 = 2 * c
        in_copy(n0 + 1, 1).start()
        in_copy(n0, 0).wait()

        @pl.when(c > 0)
        def _():
            out_copy(n0 - 2, 0).wait()

        compute(0)
        out_copy(n0, 0).start()

        @pl.when(c < NPAIR - 1)
        def _():
            in_copy(n0 + 2, 0).start()

        in_copy(n0 + 1, 1).wait()

        @pl.when(c > 0)
        def _():
            out_copy(n0 - 1, 1).wait()

        compute(1)
        out_copy(n0 + 1, 1).start()
        return carry

    lax.fori_loop(0, NPAIR, pair_body, 0)
    out_copy(NCHUNK - 2, 0).wait()
    out_copy(NCHUNK - 1, 1).wait()


def kernel(input, perm):
    perm = perm.astype(jnp.int32)
    pidx = (perm // 128) * 1024 + (perm % 128)
    in_phys = input.reshape(ROWS // 8, 8, CH // 128, 128)
    in_phys = in_phys.transpose(0, 2, 1, 3).reshape(-1)
    out_phys = _permute(in_phys, pidx)
    out = out_phys.reshape(ROWS // 8, CH // 128, 8, 128)
    return out.transpose(0, 2, 1, 3).reshape(ROWS, CH)


# speedup vs baseline: 6.5482x; 3.0947x over previous
import functools

import jax
import jax.numpy as jnp
from jax import lax
from jax.experimental import pallas as pl
from jax.experimental.pallas import tpu as pltpu
from jax.experimental.pallas import tpu_sc as plsc

ROWS = 32768
CH = 2048
L = 16
NC = 2
NS = 16
NW = NC * NS
ROWS_PER_W = ROWS // NW
R = 8
NCHUNK = ROWS_PER_W // R
NPAIR = NCHUNK // 2
G = CH // L
CB = R * CH

_mesh = plsc.VectorSubcoreMesh(core_axis_name="c", subcore_axis_name="s")


@functools.partial(
    pl.kernel,
    mesh=_mesh,
    out_type=jax.ShapeDtypeStruct((ROWS * CH,), jnp.float32),
    scratch_types=[
        pltpu.VMEM((CH,), jnp.int32),
        pltpu.VMEM((CB,), jnp.float32),
        pltpu.VMEM((CB,), jnp.float32),
        pltpu.VMEM((CB,), jnp.float32),
        pltpu.VMEM((CB,), jnp.float32),
        pltpu.SemaphoreType.DMA,
        pltpu.SemaphoreType.DMA,
        pltpu.SemaphoreType.DMA,
        pltpu.SemaphoreType.DMA,
    ],
    compiler_params=pltpu.CompilerParams(needs_layout_passes=False),
)
def _permute(in_hbm, pidx_hbm, out_hbm, pidx_v,
             in0, in1, out0, out1, si0, si1, so0, so1):
    wid = lax.axis_index("s") * NC + lax.axis_index("c")
    base = wid * ROWS_PER_W * CH
    pltpu.sync_copy(pidx_hbm, pidx_v)

    ins, outs = [in0, in1], [out0, out1]
    sis, sos = [si0, si1], [so0, so1]

    def in_copy(n, b):
        return pltpu.make_async_copy(
            in_hbm.at[pl.ds(base + n * CB, CB)], ins[b], sis[b])

    def out_copy(n, b):
        return pltpu.make_async_copy(
            outs[b], out_hbm.at[pl.ds(base + n * CB, CB)], sos[b])

    def compute(b):
        inbuf, outbuf = ins[b], outs[b]

        @plsc.parallel_loop(0, G, unroll=2)
        def _group_body(g):
            q0 = (g >> 3) * 1024 + (g & 7) * L
            idx = pidx_v[pl.ds(g * L, L)]
            for r in range(R):
                v = plsc.load_gather(inbuf, [idx + (r * 128)])
                outbuf[pl.ds(q0 + r * 128, L)] = v

    in_copy(0, 0).start()

    def pair_body(c, carry):
        n0 = 2 * c
        in_copy(n0 + 1, 1).start()
        in_copy(n0, 0).wait()

        @pl.when(c > 0)
        def _():
            out_copy(n0 - 2, 0).wait()

        compute(0)
        out_copy(n0, 0).start()

        @pl.when(c < NPAIR - 1)
        def _():
            in_copy(n0 + 2, 0).start()

        in_copy(n0 + 1, 1).wait()

        @pl.when(c > 0)
        def _():
            out_copy(n0 - 1, 1).wait()

        compute(1)
        out_copy(n0 + 1, 1).start()
        return carry

    lax.fori_loop(0, NPAIR, pair_body, 0)
    out_copy(NCHUNK - 2, 0).wait()
    out_copy(NCHUNK - 1, 1).wait()


def kernel(input, perm):
    perm = perm.astype(jnp.int32)
    pidx = (perm // 128) * 1024 + (perm % 128)
    in_phys = input.reshape(ROWS // 8, 8, CH // 128, 128)
    in_phys = in_phys.transpose(0, 2, 1, 3).reshape(-1)
    out_phys = _permute(in_phys, pidx)
    out = out_phys.reshape(ROWS // 8, CH // 128, 8, 128)
    return out.transpose(0, 2, 1, 3).reshape(ROWS, CH)
